# Initial kernel scaffold; baseline (speedup 1.0000x reference)
import jax
import jax.numpy as jnp
from jax.experimental import pallas as pl


def kernel(x, adj, W, b, mlp_W, mlp_b):
    raise NotImplementedError("write your pallas kernel here")


# speedup vs baseline: 2.9124x; 2.9124x over previous
import functools

import jax
import jax.numpy as jnp
from jax import lax
from jax.experimental import pallas as pl
from jax.experimental.pallas import tpu as pltpu
from jax.experimental.pallas import tpu_sc as plsc

N_NODES = 10000
R_ACC = 10240
STRIPE = R_ACC // 16
CHUNK = 128
NC, NS = 2, 16
NW = NC * NS


def _sc_aggregate_body(x_hbm, src_hbm, dst_hbm, zeros_hbm, out_hbm,
                       src_v, dst_v, msg_v, agg, sem, k_chunks):
    cid = lax.axis_index("c")
    sid = lax.axis_index("s")
    wid = sid * NC + cid

    pltpu.sync_copy(zeros_hbm, agg.at[pl.ds(sid * STRIPE, STRIPE)])
    pltpu.sync_copy(src_hbm.at[pl.ds(wid * k_chunks, k_chunks)], src_v)
    pltpu.sync_copy(dst_hbm.at[pl.ds(wid * k_chunks, k_chunks)], dst_v)
    plsc.subcore_barrier()

    def step(j, carry):
        pltpu.async_copy(x_hbm.at[src_v.at[j]], msg_v, sem).wait()
        pltpu.sync_copy(msg_v, agg.at[dst_v.at[j]], add=True)
        return carry

    lax.fori_loop(0, k_chunks, step, 0)
    plsc.subcore_barrier()

    off = cid * R_ACC + sid * STRIPE
    pltpu.sync_copy(agg.at[pl.ds(sid * STRIPE, STRIPE)],
                    out_hbm.at[pl.ds(off, STRIPE)])


def _sc_aggregate(x, src2, dst2, zeros, k_chunks):
    mesh = plsc.VectorSubcoreMesh(core_axis_name="c", subcore_axis_name="s",
                                  num_cores=NC, num_subcores=NS)
    body = functools.partial(_sc_aggregate_body, k_chunks=k_chunks)
    return pl.kernel(
        body,
        out_type=jax.ShapeDtypeStruct((NC * R_ACC, 128), jnp.float32),
        mesh=mesh,
        scratch_types=[
            pltpu.VMEM((k_chunks, CHUNK), jnp.int32),
            pltpu.VMEM((k_chunks, CHUNK), jnp.int32),
            pltpu.VMEM((CHUNK, 128), jnp.float32),
            pltpu.VMEM_SHARED((R_ACC, 128), jnp.float32),
            pltpu.SemaphoreType.DMA,
        ],
    )(x, src2, dst2, zeros)


def _tc_head_body(p_ref, w_ref, b_ref, mw_ref, mb_ref, o_ref):
    s = p_ref[0] + p_ref[1]
    h = jnp.dot(s, w_ref[...], preferred_element_type=jnp.float32,
                precision=lax.Precision.HIGHEST)
    h = jnp.maximum(h + b_ref[...], 0.0)
    o = lax.dot_general(h, mw_ref[...], (((1,), (1,)), ((), ())),
                        preferred_element_type=jnp.float32,
                        precision=lax.Precision.HIGHEST)
    o_ref[...] = o + mb_ref[...]


def _tc_head(partials, W, b, mlp_W, mlp_b):
    blk = 400
    grid = (N_NODES // blk,)
    return pl.pallas_call(
        _tc_head_body,
        grid=grid,
        in_specs=[
            pl.BlockSpec((2, blk, 128), lambda i: (0, i, 0)),
            pl.BlockSpec((128, 128), lambda i: (0, 0)),
            pl.BlockSpec((1, 128), lambda i: (0, 0)),
            pl.BlockSpec((64, 128), lambda i: (0, 0)),
            pl.BlockSpec((1, 64), lambda i: (0, 0)),
        ],
        out_specs=pl.BlockSpec((blk, 64), lambda i: (i, 0)),
        out_shape=jax.ShapeDtypeStruct((N_NODES, 64), jnp.float32),
    )(partials, W, b, mlp_W, mlp_b)


def kernel(x, adj, W, b, mlp_W, mlp_b):
    src = adj[0]
    dst = adj[1]
    e = src.shape[0]
    k_chunks = -(-e // (CHUNK * NW))
    k_chunks = -(-k_chunks // 8) * 8
    e_pad = NW * k_chunks * CHUNK
    pad = e_pad - e
    src_p = jnp.concatenate([src, jnp.zeros((pad,), jnp.int32)])
    dst_p = jnp.concatenate([dst, jnp.full((pad,), N_NODES, jnp.int32)])
    src2 = src_p.reshape(NW * k_chunks, CHUNK)
    dst2 = dst_p.reshape(NW * k_chunks, CHUNK)
    zeros = jnp.zeros((STRIPE, 128), jnp.float32)

    partials = _sc_aggregate(x, src2, dst2, zeros, k_chunks)
    partials = partials.reshape(NC, R_ACC, 128)
    return _tc_head(partials, W, b.reshape(1, 128), mlp_W, mlp_b.reshape(1, 64))


# speedup vs baseline: 2.9853x; 1.0250x over previous
import functools

import jax
import jax.numpy as jnp
from jax import lax
from jax.experimental import pallas as pl
from jax.experimental.pallas import tpu as pltpu
from jax.experimental.pallas import tpu_sc as plsc

N_NODES = 10000
R_ACC = 10240
STRIPE = R_ACC // 16
CHUNK = 128
NC, NS = 2, 16
NW = NC * NS


NBUF = 2
SB = 16


def _sc_aggregate_body(x_hbm, src_hbm, dst_hbm, zeros_hbm, out_hbm,
                       src_v, dst_v, msg_v, agg, gsems, ssems, k_chunks):
    cid = lax.axis_index("c")
    sid = lax.axis_index("s")
    wid = sid * NC + cid

    pltpu.sync_copy(zeros_hbm, agg.at[pl.ds(sid * STRIPE, STRIPE)])
    plsc.subcore_barrier()

    def fire_gather(c, b):
        pltpu.async_copy(x_hbm.at[src_v.at[c]], msg_v.at[b], gsems[b])

    def superblock(s, carry):
        off = wid * k_chunks + s * SB
        pltpu.sync_copy(src_hbm.at[pl.ds(off, SB)], src_v)
        pltpu.sync_copy(dst_hbm.at[pl.ds(off, SB)], dst_v)
        for b in range(NBUF):
            fire_gather(b, b)

        def group(g, carry2):
            base = g * NBUF
            for b in range(NBUF):
                pltpu.make_async_copy(x_hbm.at[src_v.at[base + b]],
                                      msg_v.at[b], gsems[b]).wait()
            for b in range(NBUF):
                pltpu.async_copy(msg_v.at[b], agg.at[dst_v.at[base + b]],
                                 ssems[b], add=True)
            for b in range(NBUF):
                pltpu.make_async_copy(msg_v.at[b], agg.at[dst_v.at[base + b]],
                                      ssems[b]).wait()
            @pl.when(g < SB // NBUF - 1)
            def _():
                for b in range(NBUF):
                    fire_gather(base + NBUF + b, b)
            return carry2

        lax.fori_loop(0, SB // NBUF, group, 0)
        return carry

    lax.fori_loop(0, k_chunks // SB, superblock, 0)
    plsc.subcore_barrier()

    off = cid * R_ACC + sid * STRIPE
    pltpu.sync_copy(agg.at[pl.ds(sid * STRIPE, STRIPE)],
                    out_hbm.at[pl.ds(off, STRIPE)])


def _sc_aggregate(x, src2, dst2, zeros, k_chunks):
    mesh = plsc.VectorSubcoreMesh(core_axis_name="c", subcore_axis_name="s",
                                  num_cores=NC, num_subcores=NS)
    body = functools.partial(_sc_aggregate_body, k_chunks=k_chunks)
    return pl.kernel(
        body,
        out_type=jax.ShapeDtypeStruct((NC * R_ACC, 128), jnp.float32),
        mesh=mesh,
        scratch_types=[
            pltpu.VMEM((SB, CHUNK), jnp.int32),
            pltpu.VMEM((SB, CHUNK), jnp.int32),
            pltpu.VMEM((NBUF, CHUNK, 128), jnp.float32),
            pltpu.VMEM_SHARED((R_ACC, 128), jnp.float32),
            [pltpu.SemaphoreType.DMA] * NBUF,
            [pltpu.SemaphoreType.DMA] * NBUF,
        ],
    )(x, src2, dst2, zeros)


def _tc_head_body(p_ref, w_ref, b_ref, mw_ref, mb_ref, o_ref):
    s = p_ref[0] + p_ref[1]
    h = jnp.dot(s, w_ref[...], preferred_element_type=jnp.float32,
                precision=lax.Precision.HIGHEST)
    h = jnp.maximum(h + b_ref[...], 0.0)
    o = lax.dot_general(h, mw_ref[...], (((1,), (1,)), ((), ())),
                        preferred_element_type=jnp.float32,
                        precision=lax.Precision.HIGHEST)
    o_ref[...] = o + mb_ref[...]


def _tc_head(partials, W, b, mlp_W, mlp_b):
    blk = 400
    grid = (N_NODES // blk,)
    return pl.pallas_call(
        _tc_head_body,
        grid=grid,
        in_specs=[
            pl.BlockSpec((2, blk, 128), lambda i: (0, i, 0)),
            pl.BlockSpec((128, 128), lambda i: (0, 0)),
            pl.BlockSpec((1, 128), lambda i: (0, 0)),
            pl.BlockSpec((64, 128), lambda i: (0, 0)),
            pl.BlockSpec((1, 64), lambda i: (0, 0)),
        ],
        out_specs=pl.BlockSpec((blk, 64), lambda i: (i, 0)),
        out_shape=jax.ShapeDtypeStruct((N_NODES, 64), jnp.float32),
    )(partials, W, b, mlp_W, mlp_b)


def kernel(x, adj, W, b, mlp_W, mlp_b):
    src = adj[0]
    dst = adj[1]
    e = src.shape[0]
    k_chunks = -(-e // (CHUNK * NW))
    k_chunks = -(-k_chunks // 8) * 8
    e_pad = NW * k_chunks * CHUNK
    pad = e_pad - e
    src_p = jnp.concatenate([src, jnp.zeros((pad,), jnp.int32)])
    dst_p = jnp.concatenate([dst, jnp.full((pad,), N_NODES, jnp.int32)])
    src2 = src_p.reshape(NW * k_chunks, CHUNK)
    dst2 = dst_p.reshape(NW * k_chunks, CHUNK)
    zeros = jnp.zeros((STRIPE, 128), jnp.float32)

    partials = _sc_aggregate(x, src2, dst2, zeros, k_chunks)
    partials = partials.reshape(NC, R_ACC, 128)
    return _tc_head(partials, W, b.reshape(1, 128), mlp_W, mlp_b.reshape(1, 64))


# speedup vs baseline: 8.7553x; 2.9328x over previous
import functools

import jax
import jax.numpy as jnp
from jax import lax
from jax.experimental import pallas as pl
from jax.experimental.pallas import tpu as pltpu
from jax.experimental.pallas import tpu_sc as plsc

N_NODES = 10000
R_ACC = 10240
STRIPE = R_ACC // 16
CHUNK = 128
NC, NS = 2, 16
NW = NC * NS


NBUF = 2
SB = 16


def _sc_aggregate_body(x_hbm, src_hbm, dst_hbm, zeros_hbm, out_hbm,
                       src_v, dst_v, msg_v, agg, gsems, ssems, k_chunks):
    cid = lax.axis_index("c")
    sid = lax.axis_index("s")
    wid = sid * NC + cid

    pltpu.sync_copy(zeros_hbm, agg.at[pl.ds(sid * STRIPE, STRIPE)])
    plsc.subcore_barrier()

    def fire_gather(c, b):
        pltpu.async_copy(x_hbm.at[src_v.at[c]], msg_v.at[b], gsems[b])

    def superblock(s, carry):
        off = wid * k_chunks + s * SB
        pltpu.sync_copy(src_hbm.at[pl.ds(off, SB)], src_v)
        pltpu.sync_copy(dst_hbm.at[pl.ds(off, SB)], dst_v)
        for b in range(NBUF):
            fire_gather(b, b)

        def group(g, carry2):
            base = g * NBUF
            for b in range(NBUF):
                pltpu.make_async_copy(x_hbm.at[src_v.at[base + b]],
                                      msg_v.at[b], gsems[b]).wait()
            for b in range(NBUF):
                pltpu.async_copy(msg_v.at[b], agg.at[dst_v.at[base + b]],
                                 ssems[b], add=True)
            for b in range(NBUF):
                pltpu.make_async_copy(msg_v.at[b], agg.at[dst_v.at[base + b]],
                                      ssems[b]).wait()
            @pl.when(g < SB // NBUF - 1)
            def _():
                for b in range(NBUF):
                    fire_gather(base + NBUF + b, b)
            return carry2

        lax.fori_loop(0, SB // NBUF, group, 0)
        return carry

    lax.fori_loop(0, k_chunks // SB, superblock, 0)
    plsc.subcore_barrier()

    off = cid * R_ACC + sid * STRIPE
    pltpu.sync_copy(agg.at[pl.ds(sid * STRIPE, STRIPE)],
                    out_hbm.at[pl.ds(off, STRIPE)])


def _sc_aggregate(x, src2, dst2, zeros, k_chunks):
    mesh = plsc.VectorSubcoreMesh(core_axis_name="c", subcore_axis_name="s",
                                  num_cores=NC, num_subcores=NS)
    body = functools.partial(_sc_aggregate_body, k_chunks=k_chunks)
    return pl.kernel(
        body,
        out_type=jax.ShapeDtypeStruct((NC * R_ACC, 128), jnp.float32),
        mesh=mesh,
        scratch_types=[
            pltpu.VMEM((SB, CHUNK), jnp.int32),
            pltpu.VMEM((SB, CHUNK), jnp.int32),
            pltpu.VMEM((NBUF, CHUNK, 128), jnp.float32),
            pltpu.VMEM_SHARED((R_ACC, 128), jnp.float32),
            [pltpu.SemaphoreType.DMA] * NBUF,
            [pltpu.SemaphoreType.DMA] * NBUF,
        ],
    )(x, src2, dst2, zeros)


def _tc_head_body(p_ref, w_ref, b_ref, mw_ref, mb_ref, o_ref):
    s = p_ref[0] + p_ref[1]
    h = jnp.dot(s, w_ref[...], preferred_element_type=jnp.float32,
                precision=lax.Precision.HIGHEST)
    h = jnp.maximum(h + b_ref[...], 0.0)
    o = lax.dot_general(h, mw_ref[...], (((1,), (1,)), ((), ())),
                        preferred_element_type=jnp.float32,
                        precision=lax.Precision.HIGHEST)
    o_ref[...] = o + mb_ref[...]


def _tc_head(partials, W, b, mlp_W, mlp_b):
    blk = 400
    grid = (N_NODES // blk,)
    return pl.pallas_call(
        _tc_head_body,
        grid=grid,
        in_specs=[
            pl.BlockSpec((2, blk, 128), lambda i: (0, i, 0)),
            pl.BlockSpec((128, 128), lambda i: (0, 0)),
            pl.BlockSpec((1, 128), lambda i: (0, 0)),
            pl.BlockSpec((64, 128), lambda i: (0, 0)),
            pl.BlockSpec((1, 64), lambda i: (0, 0)),
        ],
        out_specs=pl.BlockSpec((blk, 64), lambda i: (i, 0)),
        out_shape=jax.ShapeDtypeStruct((N_NODES, 64), jnp.float32),
    )(partials, W, b, mlp_W, mlp_b)


def kernel(x, adj, W, b, mlp_W, mlp_b):
    src = adj[0]
    dst = adj[1]
    e = src.shape[0]
    k_chunks = -(-e // (CHUNK * NW))
    k_chunks = -(-k_chunks // 8) * 8
    e_pad = NW * k_chunks * CHUNK
    pad = e_pad - e
    pad_i = jnp.arange(pad, dtype=jnp.int32)
    src_p = jnp.concatenate([src, pad_i % N_NODES])
    dst_p = jnp.concatenate([dst, N_NODES + pad_i % (R_ACC - N_NODES)])
    src2 = src_p.reshape(NW * k_chunks, CHUNK)
    dst2 = dst_p.reshape(NW * k_chunks, CHUNK)
    zeros = jnp.zeros((STRIPE, 128), jnp.float32)

    partials = _sc_aggregate(x, src2, dst2, zeros, k_chunks)
    partials = partials.reshape(NC, R_ACC, 128)
    return _tc_head(partials, W, b.reshape(1, 128), mlp_W, mlp_b.reshape(1, 64))


# speedup vs baseline: 9.6801x; 1.1056x over previous
import functools

import jax
import jax.numpy as jnp
from jax import lax
from jax.experimental import pallas as pl
from jax.experimental.pallas import tpu as pltpu
from jax.experimental.pallas import tpu_sc as plsc

N_NODES = 10000
R_ACC = 10240
STRIPE = R_ACC // 16
CHUNK = 128
NC, NS = 2, 16
NW = NC * NS
SB = 16


def _sc_aggregate_body(x_hbm, src_hbm, dst_hbm, zeros_hbm, out_hbm,
                       src_v, dst_v, msg_v, agg, gsems, ssems, k_chunks):
    cid = lax.axis_index("c")
    sid = lax.axis_index("s")
    wid = sid * NC + cid

    pltpu.sync_copy(zeros_hbm, agg.at[pl.ds(sid * STRIPE, STRIPE)])
    plsc.subcore_barrier()

    def slab_load(j, slot):
        off = wid * k_chunks + j * SB
        pltpu.sync_copy(src_hbm.at[pl.ds(off, SB)], src_v.at[slot])
        pltpu.sync_copy(dst_hbm.at[pl.ds(off, SB)], dst_v.at[slot])

    def idx_ref(ref, c):
        return ref.at[(c // SB) % 2, c % SB]

    def fire_gather(c, b):
        pltpu.async_copy(x_hbm.at[idx_ref(src_v, c)], msg_v.at[b], gsems[b])

    def wait_gather(c, b):
        pltpu.make_async_copy(x_hbm.at[idx_ref(src_v, c)],
                              msg_v.at[b], gsems[b]).wait()

    def fire_scatter(c, b):
        pltpu.async_copy(msg_v.at[b], agg.at[idx_ref(dst_v, c)],
                         ssems[b], add=True)

    def wait_scatter(c, b):
        pltpu.make_async_copy(msg_v.at[b], agg.at[idx_ref(dst_v, c)],
                              ssems[b]).wait()

    n_pairs = k_chunks // 2
    slab_load(0, 0)
    fire_gather(0, 0)

    def pair(p, carry):
        c0 = 2 * p
        c1 = c0 + 1
        wait_gather(c0, 0)
        fire_scatter(c0, 0)

        @pl.when(p > 0)
        def _():
            wait_scatter(c0 - 1, 1)

        fire_gather(c1, 1)
        wait_gather(c1, 1)
        fire_scatter(c1, 1)
        wait_scatter(c0, 0)

        @pl.when(jnp.logical_and(p % (SB // 2) == SB // 2 - 1,
                                 p < n_pairs - 1))
        def _():
            j = p // (SB // 2) + 1
            slab_load(j, j % 2)

        @pl.when(p < n_pairs - 1)
        def _():
            fire_gather(c0 + 2, 0)

        return carry

    lax.fori_loop(0, n_pairs, pair, 0)
    wait_scatter(k_chunks - 1, 1)
    plsc.subcore_barrier()

    off = cid * R_ACC + sid * STRIPE
    pltpu.sync_copy(agg.at[pl.ds(sid * STRIPE, STRIPE)],
                    out_hbm.at[pl.ds(off, STRIPE)])


def _sc_aggregate(x, src2, dst2, zeros, k_chunks):
    mesh = plsc.VectorSubcoreMesh(core_axis_name="c", subcore_axis_name="s",
                                  num_cores=NC, num_subcores=NS)
    body = functools.partial(_sc_aggregate_body, k_chunks=k_chunks)
    return pl.kernel(
        body,
        out_type=jax.ShapeDtypeStruct((NC * R_ACC, 128), jnp.float32),
        mesh=mesh,
        scratch_types=[
            pltpu.VMEM((2, SB, CHUNK), jnp.int32),
            pltpu.VMEM((2, SB, CHUNK), jnp.int32),
            pltpu.VMEM((2, CHUNK, 128), jnp.float32),
            pltpu.VMEM_SHARED((R_ACC, 128), jnp.float32),
            [pltpu.SemaphoreType.DMA] * 2,
            [pltpu.SemaphoreType.DMA] * 2,
        ],
    )(x, src2, dst2, zeros)


def _tc_head_body(p_ref, w_ref, b_ref, mw_ref, mb_ref, o_ref):
    s = p_ref[0] + p_ref[1]
    h = jnp.dot(s, w_ref[...], preferred_element_type=jnp.float32,
                precision=lax.Precision.HIGHEST)
    h = jnp.maximum(h + b_ref[...], 0.0)
    o = lax.dot_general(h, mw_ref[...], (((1,), (1,)), ((), ())),
                        preferred_element_type=jnp.float32,
                        precision=lax.Precision.HIGHEST)
    o_ref[...] = o + mb_ref[...]


def _tc_head(partials, W, b, mlp_W, mlp_b):
    blk = 400
    grid = (N_NODES // blk,)
    return pl.pallas_call(
        _tc_head_body,
        grid=grid,
        in_specs=[
            pl.BlockSpec((2, blk, 128), lambda i: (0, i, 0)),
            pl.BlockSpec((128, 128), lambda i: (0, 0)),
            pl.BlockSpec((1, 128), lambda i: (0, 0)),
            pl.BlockSpec((64, 128), lambda i: (0, 0)),
            pl.BlockSpec((1, 64), lambda i: (0, 0)),
        ],
        out_specs=pl.BlockSpec((blk, 64), lambda i: (i, 0)),
        out_shape=jax.ShapeDtypeStruct((N_NODES, 64), jnp.float32),
    )(partials, W, b, mlp_W, mlp_b)


def kernel(x, adj, W, b, mlp_W, mlp_b):
    src = adj[0]
    dst = adj[1]
    e = src.shape[0]
    k_chunks = -(-e // (CHUNK * NW))
    k_chunks = -(-k_chunks // 8) * 8
    e_pad = NW * k_chunks * CHUNK
    pad = e_pad - e
    pad_i = jnp.arange(pad, dtype=jnp.int32)
    src_p = jnp.concatenate([src, pad_i % N_NODES])
    dst_p = jnp.concatenate([dst, N_NODES + pad_i % (R_ACC - N_NODES)])
    src2 = src_p.reshape(NW * k_chunks, CHUNK)
    dst2 = dst_p.reshape(NW * k_chunks, CHUNK)
    zeros = jnp.zeros((STRIPE, 128), jnp.float32)

    partials = _sc_aggregate(x, src2, dst2, zeros, k_chunks)
    partials = partials.reshape(NC, R_ACC, 128)
    return _tc_head(partials, W, b.reshape(1, 128), mlp_W, mlp_b.reshape(1, 64))
